# baseline (speedup 1.0000x reference)
import functools

import jax
import jax.numpy as jnp
from jax import lax
from jax.experimental import pallas as pl
from jax.experimental.pallas import tpu as pltpu
from jax.experimental.pallas import tpu_sc as plsc

L = 16
NC = 2
NS = 16
NW = NC * NS
BATCH_SIZE = 16384
N_FACT = 64
CHUNK = 128
N_CHUNKS = BATCH_SIZE // NW // CHUNK
ROWS_PER_W = CHUNK * N_CHUNKS


def _hsum16(vecs):
    iota = lax.iota(jnp.int32, L)
    cur = list(vecs)
    h = L // 2
    while h >= 1:
        perm = iota ^ h
        mask = (iota & h) == 0
        half = len(cur) // 2
        nxt = []
        for k in range(half):
            x, y = cur[k], cur[k + half]
            x2 = x + x.at[perm].get(mode="promise_in_bounds", unique_indices=True)
            y2 = y + y.at[perm].get(mode="promise_in_bounds", unique_indices=True)
            nxt.append(jnp.where(mask, x2, y2))
        cur = nxt
        h //= 2
    return cur[0]


def _sc_kernel(uf_hbm, if_hbm, uidx_hbm, iidx_hbm, out_hbm,
               uidx_v, iidx_v,
               ub0, ub1, ub2, ub3,
               vb0, vb1, vb2, vb3,
               out_v,
               su0, su1, su2, su3,
               sv0, sv1, sv2, sv3):
    wid = lax.axis_index("s") * NC + lax.axis_index("c")
    base = wid * ROWS_PER_W

    for c in range(N_CHUNKS):
        pltpu.sync_copy(uidx_hbm.at[pl.ds(base + c * CHUNK, CHUNK)],
                        uidx_v.at[c])
        pltpu.sync_copy(iidx_hbm.at[pl.ds(base + c * CHUNK, CHUNK)],
                        iidx_v.at[c])

    ubufs = [ub0, ub1, ub2, ub3]
    vbufs = [vb0, vb1, vb2, vb3]
    usems = [su0, su1, su2, su3]
    vsems = [sv0, sv1, sv2, sv3]

    ucps = []
    vcps = []
    for c in range(N_CHUNKS):
        ucps.append(pltpu.async_copy(uf_hbm.at[uidx_v.at[c]], ubufs[c], usems[c]))
        vcps.append(pltpu.async_copy(if_hbm.at[iidx_v.at[c]], vbufs[c], vsems[c]))

    for c in range(N_CHUNKS):
        ucps[c].wait()
        vcps[c].wait()
        ub = ubufs[c]
        vb = vbufs[c]

        def body(g, _, ub=ub, vb=vb, c=c):
            partials = []
            for k in range(L):
                r = g * L + k
                acc = ub[r, pl.ds(0, L)] * vb[r, pl.ds(0, L)]
                for j in range(1, N_FACT // L):
                    acc = acc + ub[r, pl.ds(j * L, L)] * vb[r, pl.ds(j * L, L)]
                partials.append(acc)
            out_v[pl.ds(c * CHUNK + g * L, L)] = _hsum16(partials)
            return 0

        lax.fori_loop(0, CHUNK // L, body, 0)

    pltpu.sync_copy(out_v, out_hbm.at[pl.ds(base, ROWS_PER_W)])


@functools.partial(
    pl.kernel,
    out_type=jax.ShapeDtypeStruct((BATCH_SIZE,), jnp.float32),
    mesh=plsc.VectorSubcoreMesh(core_axis_name="c", subcore_axis_name="s"),
    compiler_params=pltpu.CompilerParams(use_tc_tiling_on_sc=False),
    scratch_types=(
        [pltpu.VMEM((N_CHUNKS, CHUNK), jnp.int32)] * 2
        + [pltpu.VMEM((CHUNK, N_FACT), jnp.float32)] * (2 * N_CHUNKS)
        + [pltpu.VMEM((ROWS_PER_W,), jnp.float32)]
        + [pltpu.SemaphoreType.DMA] * (2 * N_CHUNKS)
    ),
)
def _mf_dot(uf_hbm, if_hbm, uidx_hbm, iidx_hbm, out_hbm, *scratch):
    _sc_kernel(uf_hbm, if_hbm, uidx_hbm, iidx_hbm, out_hbm, *scratch)


def kernel(user, item, user_factors, item_factors):
    return _mf_dot(user_factors, item_factors,
                   user.astype(jnp.int32), item.astype(jnp.int32))


# speedup vs baseline: 1.1538x; 1.1538x over previous
import functools

import jax
import jax.numpy as jnp
from jax import lax
from jax.experimental import pallas as pl
from jax.experimental.pallas import tpu as pltpu
from jax.experimental.pallas import tpu_sc as plsc

L = 16
NC = 2
NS = 16
NW = NC * NS
BATCH_SIZE = 16384
N_FACT = 64
N_ROWS = 1000000
PAIR_K = 1 << 19
PAIR_ROWS = PAIR_K
CHUNK = 64
N_CHUNKS = BATCH_SIZE // NW // CHUNK
ROWS_PER_W = CHUNK * N_CHUNKS

TWIN = 1024
TSTEPS = PAIR_K // TWIN


def _transpose_body(lo_ref, hi_ref, out_ref):
    out_ref[:, pl.ds(0, N_FACT)] = lo_ref[...].T
    out_ref[:, pl.ds(N_FACT, N_FACT)] = hi_ref[...].T


def _repack(table_t):
    return pl.pallas_call(
        _transpose_body,
        grid=(TSTEPS,),
        in_specs=[pl.BlockSpec((N_FACT, TWIN), lambda i: (0, i)),
                  pl.BlockSpec((N_FACT, TWIN),
                               lambda i: (0, jnp.minimum(i + TSTEPS,
                                                         -(-N_ROWS // TWIN) - 1)))],
        out_specs=pl.BlockSpec((TWIN, 2 * N_FACT), lambda i: (i, 0)),
        out_shape=jax.ShapeDtypeStruct((PAIR_ROWS, 2 * N_FACT), jnp.float32),
    )(table_t, table_t)


def _hsum16(vecs):
    iota = lax.iota(jnp.int32, L)
    cur = list(vecs)
    h = L // 2
    while h >= 1:
        perm = iota ^ h
        mask = (iota & h) == 0
        half = len(cur) // 2
        nxt = []
        for k in range(half):
            x, y = cur[k], cur[k + half]
            x2 = x + x.at[perm].get(mode="promise_in_bounds", unique_indices=True)
            y2 = y + y.at[perm].get(mode="promise_in_bounds", unique_indices=True)
            nxt.append(jnp.where(mask, x2, y2))
        cur = nxt
        h //= 2
    return cur[0]


def _sc_kernel(uf_hbm, if_hbm, uidx_hbm, iidx_hbm, out_hbm,
               uraw_v, iraw_v, uhlf_v, ihlf_v,
               ub0, ub1, vb0, vb1, out_v,
               su0, su1, sv0, sv1):
    wid = lax.axis_index("s") * NC + lax.axis_index("c")
    base = wid * ROWS_PER_W

    for c in range(N_CHUNKS):
        pltpu.sync_copy(uidx_hbm.at[pl.ds(base + c * CHUNK, CHUNK)],
                        uraw_v.at[c])
        pltpu.sync_copy(iidx_hbm.at[pl.ds(base + c * CHUNK, CHUNK)],
                        iraw_v.at[c])

    for c in range(N_CHUNKS):
        for s in range(CHUNK // L):
            uhlf_v[c, pl.ds(s * L, L)] = uraw_v[c, pl.ds(s * L, L)] & (PAIR_K - 1)
            ihlf_v[c, pl.ds(s * L, L)] = iraw_v[c, pl.ds(s * L, L)] & (PAIR_K - 1)

    ubufs = [ub0, ub1]
    vbufs = [vb0, vb1]
    usems = [su0, su1]
    vsems = [sv0, sv1]

    def fetch(c, buf):
        return (pltpu.async_copy(uf_hbm.at[uhlf_v.at[c]], ubufs[buf],
                                 usems[buf]),
                pltpu.async_copy(if_hbm.at[ihlf_v.at[c]], vbufs[buf],
                                 vsems[buf]))

    def compute(c, buf):
        ub = ubufs[buf]
        vb = vbufs[buf]

        def body(g, _):
            uraw = uraw_v[c, pl.ds(g * L, L)]
            iraw = iraw_v[c, pl.ds(g * L, L)]
            partials = []
            for k in range(L):
                r = g * L + k
                pu = (uraw[k] >> 19) != 0
                pv = (iraw[k] >> 19) != 0
                acc = None
                for j in range(N_FACT // L):
                    su = jnp.where(pu, ub[r, pl.ds(N_FACT + j * L, L)],
                                   ub[r, pl.ds(j * L, L)])
                    sv = jnp.where(pv, vb[r, pl.ds(N_FACT + j * L, L)],
                                   vb[r, pl.ds(j * L, L)])
                    a = su * sv
                    acc = a if acc is None else acc + a
                partials.append(acc)
            out_v[pl.ds(c * CHUNK + g * L, L)] = _hsum16(partials)
            return 0

        lax.fori_loop(0, CHUNK // L, body, 0)

    pend = {0: fetch(0, 0), 1: None}
    for c in range(N_CHUNKS):
        buf = c % 2
        if c + 1 < N_CHUNKS:
            pend[1 - buf] = fetch(c + 1, 1 - buf)
        for cp in pend[buf]:
            cp.wait()
        compute(c, buf)

    pltpu.sync_copy(out_v, out_hbm.at[pl.ds(base, ROWS_PER_W)])


@functools.partial(
    pl.kernel,
    out_type=jax.ShapeDtypeStruct((BATCH_SIZE,), jnp.float32),
    mesh=plsc.VectorSubcoreMesh(core_axis_name="c", subcore_axis_name="s"),
    compiler_params=pltpu.CompilerParams(use_tc_tiling_on_sc=False),
    scratch_types=(
        [pltpu.VMEM((N_CHUNKS, CHUNK), jnp.int32)] * 4
        + [pltpu.VMEM((CHUNK, 2 * N_FACT), jnp.float32)] * 4
        + [pltpu.VMEM((ROWS_PER_W,), jnp.float32)]
        + [pltpu.SemaphoreType.DMA] * 4
    ),
)
def _mf_dot(uf_hbm, if_hbm, uidx_hbm, iidx_hbm, out_hbm, *scratch):
    _sc_kernel(uf_hbm, if_hbm, uidx_hbm, iidx_hbm, out_hbm, *scratch)


def kernel(user, item, user_factors, item_factors):
    u2 = _repack(user_factors.T)
    i2 = _repack(item_factors.T)
    return _mf_dot(u2, i2, user.astype(jnp.int32), item.astype(jnp.int32))


# speedup vs baseline: 1.5712x; 1.3618x over previous
import functools

import jax
import jax.numpy as jnp
from jax import lax
from jax.experimental import pallas as pl
from jax.experimental.pallas import tpu as pltpu
from jax.experimental.pallas import tpu_sc as plsc

L = 16
NC = 2
NS = 16
NW = NC * NS
BATCH_SIZE = 16384
N_FACT = 64
N_ROWS = 1000000
PAIR_K = 1 << 19
PAIR_ROWS = PAIR_K
CHUNK = 64
N_CHUNKS = BATCH_SIZE // NW // CHUNK
ROWS_PER_W = CHUNK * N_CHUNKS

TWIN = 2048
TSTEPS = PAIR_K // TWIN


def _transpose_body(lo_ref, hi_ref, out_ref):
    eye = jnp.eye(N_FACT, dtype=jnp.float32)
    dn = (((0,), (0,)), ((), ()))
    out_ref[:, pl.ds(0, N_FACT)] = lax.dot_general(
        lo_ref[...], eye, dn, preferred_element_type=jnp.float32)
    out_ref[:, pl.ds(N_FACT, N_FACT)] = lax.dot_general(
        hi_ref[...], eye, dn, preferred_element_type=jnp.float32)


def _repack(table_t):
    return pl.pallas_call(
        _transpose_body,
        grid=(TSTEPS,),
        in_specs=[pl.BlockSpec((N_FACT, TWIN), lambda i: (0, i)),
                  pl.BlockSpec((N_FACT, TWIN),
                               lambda i: (0, jnp.minimum(i + TSTEPS,
                                                         -(-N_ROWS // TWIN) - 1)))],
        out_specs=pl.BlockSpec((TWIN, 2 * N_FACT), lambda i: (i, 0)),
        out_shape=jax.ShapeDtypeStruct((PAIR_ROWS, 2 * N_FACT), jnp.float32),
    )(table_t, table_t)


def _hsum16(vecs):
    iota = lax.iota(jnp.int32, L)
    cur = list(vecs)
    h = L // 2
    while h >= 1:
        perm = iota ^ h
        mask = (iota & h) == 0
        half = len(cur) // 2
        nxt = []
        for k in range(half):
            x, y = cur[k], cur[k + half]
            x2 = x + x.at[perm].get(mode="promise_in_bounds", unique_indices=True)
            y2 = y + y.at[perm].get(mode="promise_in_bounds", unique_indices=True)
            nxt.append(jnp.where(mask, x2, y2))
        cur = nxt
        h //= 2
    return cur[0]


def _sc_kernel(uf_hbm, if_hbm, uidx_hbm, iidx_hbm, out_hbm,
               uraw_v, iraw_v, uhlf_v, ihlf_v,
               ub0, ub1, vb0, vb1, out_v,
               su0, su1, sv0, sv1):
    wid = lax.axis_index("s") * NC + lax.axis_index("c")
    base = wid * ROWS_PER_W

    for c in range(N_CHUNKS):
        pltpu.sync_copy(uidx_hbm.at[pl.ds(base + c * CHUNK, CHUNK)],
                        uraw_v.at[c])
        pltpu.sync_copy(iidx_hbm.at[pl.ds(base + c * CHUNK, CHUNK)],
                        iraw_v.at[c])

    for c in range(N_CHUNKS):
        for s in range(CHUNK // L):
            uhlf_v[c, pl.ds(s * L, L)] = uraw_v[c, pl.ds(s * L, L)] & (PAIR_K - 1)
            ihlf_v[c, pl.ds(s * L, L)] = iraw_v[c, pl.ds(s * L, L)] & (PAIR_K - 1)

    ubufs = [ub0, ub1]
    vbufs = [vb0, vb1]
    usems = [su0, su1]
    vsems = [sv0, sv1]

    def fetch(c, buf):
        return (pltpu.async_copy(uf_hbm.at[uhlf_v.at[c]], ubufs[buf],
                                 usems[buf]),
                pltpu.async_copy(if_hbm.at[ihlf_v.at[c]], vbufs[buf],
                                 vsems[buf]))

    def compute(c, buf):
        ub = ubufs[buf]
        vb = vbufs[buf]

        def body(g, _):
            uraw = uraw_v[c, pl.ds(g * L, L)]
            iraw = iraw_v[c, pl.ds(g * L, L)]
            partials = []
            for k in range(L):
                r = g * L + k
                pu = (uraw[k] >> 19) != 0
                pv = (iraw[k] >> 19) != 0
                acc = None
                for j in range(N_FACT // L):
                    su = jnp.where(pu, ub[r, pl.ds(N_FACT + j * L, L)],
                                   ub[r, pl.ds(j * L, L)])
                    sv = jnp.where(pv, vb[r, pl.ds(N_FACT + j * L, L)],
                                   vb[r, pl.ds(j * L, L)])
                    a = su * sv
                    acc = a if acc is None else acc + a
                partials.append(acc)
            out_v[pl.ds(c * CHUNK + g * L, L)] = _hsum16(partials)
            return 0

        lax.fori_loop(0, CHUNK // L, body, 0)

    pend = {0: fetch(0, 0), 1: None}
    for c in range(N_CHUNKS):
        buf = c % 2
        if c + 1 < N_CHUNKS:
            pend[1 - buf] = fetch(c + 1, 1 - buf)
        for cp in pend[buf]:
            cp.wait()
        compute(c, buf)

    pltpu.sync_copy(out_v, out_hbm.at[pl.ds(base, ROWS_PER_W)])


@functools.partial(
    pl.kernel,
    out_type=jax.ShapeDtypeStruct((BATCH_SIZE,), jnp.float32),
    mesh=plsc.VectorSubcoreMesh(core_axis_name="c", subcore_axis_name="s"),
    compiler_params=pltpu.CompilerParams(use_tc_tiling_on_sc=False),
    scratch_types=(
        [pltpu.VMEM((N_CHUNKS, CHUNK), jnp.int32)] * 4
        + [pltpu.VMEM((CHUNK, 2 * N_FACT), jnp.float32)] * 4
        + [pltpu.VMEM((ROWS_PER_W,), jnp.float32)]
        + [pltpu.SemaphoreType.DMA] * 4
    ),
)
def _mf_dot(uf_hbm, if_hbm, uidx_hbm, iidx_hbm, out_hbm, *scratch):
    _sc_kernel(uf_hbm, if_hbm, uidx_hbm, iidx_hbm, out_hbm, *scratch)


def kernel(user, item, user_factors, item_factors):
    u2 = _repack(user_factors.T)
    i2 = _repack(item_factors.T)
    return _mf_dot(u2, i2, user.astype(jnp.int32), item.astype(jnp.int32))


# speedup vs baseline: 1.5756x; 1.0028x over previous
import functools

import jax
import jax.numpy as jnp
from jax import lax
from jax.experimental import pallas as pl
from jax.experimental.pallas import tpu as pltpu
from jax.experimental.pallas import tpu_sc as plsc

L = 16
NC = 2
NS = 16
NW = NC * NS
BATCH_SIZE = 16384
N_FACT = 64
ROWS_PER_W = BATCH_SIZE // NW
N_GROUPS = ROWS_PER_W // L


def _hsum16(vecs):
    iota = lax.iota(jnp.int32, L)
    cur = list(vecs)
    h = L // 2
    while h >= 1:
        perm = iota ^ h
        mask = (iota & h) == 0
        half = len(cur) // 2
        nxt = []
        for k in range(half):
            x, y = cur[k], cur[k + half]
            x2 = x + x.at[perm].get(mode="promise_in_bounds", unique_indices=True)
            y2 = y + y.at[perm].get(mode="promise_in_bounds", unique_indices=True)
            nxt.append(jnp.where(mask, x2, y2))
        cur = nxt
        h //= 2
    return cur[0]


def _sc_kernel(uf_hbm, if_hbm, uidx_hbm, iidx_hbm, out_hbm,
               uidx_v, iidx_v, ub, vb, out_v, sem0, sem1):
    wid = lax.axis_index("s") * NC + lax.axis_index("c")
    base = wid * ROWS_PER_W

    pltpu.sync_copy(uidx_hbm.at[pl.ds(base, ROWS_PER_W)], uidx_v)
    pltpu.sync_copy(iidx_hbm.at[pl.ds(base, ROWS_PER_W)], iidx_v)

    sems = [sem0, sem1]

    def fetch(g, buf):
        uvec = uidx_v[pl.ds(g * L, L)]
        ivec = iidx_v[pl.ds(g * L, L)]
        for k in range(L):
            pltpu.async_copy(uf_hbm.at[uvec[k]], ub.at[buf * L + k], sems[buf])
        for k in range(L):
            pltpu.async_copy(if_hbm.at[ivec[k]], vb.at[buf * L + k], sems[buf])

    def drain(buf):
        for k in range(L):
            pltpu.make_async_copy(uf_hbm.at[0], ub.at[buf * L + k],
                                  sems[buf]).wait()
        for k in range(L):
            pltpu.make_async_copy(if_hbm.at[0], vb.at[buf * L + k],
                                  sems[buf]).wait()

    def compute(g, buf):
        partials = []
        for k in range(L):
            o = buf * L + k
            acc = ub[o, pl.ds(0, L)] * vb[o, pl.ds(0, L)]
            for j in range(1, N_FACT // L):
                acc = acc + ub[o, pl.ds(j * L, L)] * vb[o, pl.ds(j * L, L)]
            partials.append(acc)
        out_v[pl.ds(g * L, L)] = _hsum16(partials)

    fetch(0, 0)

    def body(i, _):
        g0 = 2 * i
        fetch(g0 + 1, 1)
        drain(0)
        compute(g0, 0)

        @pl.when(g0 + 2 < N_GROUPS)
        def _():
            fetch(g0 + 2, 0)

        drain(1)
        compute(g0 + 1, 1)
        return 0

    lax.fori_loop(0, N_GROUPS // 2, body, 0)

    pltpu.sync_copy(out_v, out_hbm.at[pl.ds(base, ROWS_PER_W)])


@functools.partial(
    pl.kernel,
    out_type=jax.ShapeDtypeStruct((BATCH_SIZE,), jnp.float32),
    mesh=plsc.VectorSubcoreMesh(core_axis_name="c", subcore_axis_name="s"),
    compiler_params=pltpu.CompilerParams(use_tc_tiling_on_sc=True),
    scratch_types=(
        [pltpu.VMEM((ROWS_PER_W,), jnp.int32)] * 2
        + [pltpu.VMEM((2 * L, N_FACT), jnp.float32)] * 2
        + [pltpu.VMEM((ROWS_PER_W,), jnp.float32)]
        + [pltpu.SemaphoreType.DMA] * 2
    ),
)
def _mf_dot(uf_hbm, if_hbm, uidx_hbm, iidx_hbm, out_hbm, *scratch):
    _sc_kernel(uf_hbm, if_hbm, uidx_hbm, iidx_hbm, out_hbm, *scratch)


def kernel(user, item, user_factors, item_factors):
    return _mf_dot(user_factors, item_factors,
                   user.astype(jnp.int32), item.astype(jnp.int32))
